# baseline (speedup 1.0000x reference)
import functools

import jax
import jax.numpy as jnp
from jax import lax
from jax.experimental import pallas as pl
from jax.experimental.pallas import tpu as pltpu
from jax.experimental.pallas import tpu_sc as plsc

_INFO = plsc.get_sparse_core_info()
_NC = _INFO.num_cores
_NS = _INFO.num_subcores
_NW = _NC * _NS
_L = 16


@functools.partial(jax.jit, static_argnames=("b", "s", "dim"))
def _sc_gather(idx_t, weight, b, s, dim):
    bpw = b // _NW
    mesh = plsc.VectorSubcoreMesh(core_axis_name="c", subcore_axis_name="s")

    nbuf = 5
    pref = 3
    assert s % nbuf == 0 and dim % (_L * 4) == 0 and bpw % _L == 0

    @functools.partial(
        pl.kernel,
        out_type=jax.ShapeDtypeStruct((s, dim, b), jnp.float32),
        mesh=mesh,
        scratch_types=[
            pltpu.VMEM((s, bpw), jnp.int32),
            pltpu.VMEM((nbuf, bpw, dim), jnp.float32),
            pltpu.VMEM((nbuf, dim, bpw), jnp.float32),
            pltpu.SemaphoreType.DMA,
            pltpu.SemaphoreType.DMA,
        ],
        compiler_params=pltpu.CompilerParams(
            use_tc_tiling_on_sc=False, needs_layout_passes=False
        ),
    )
    def k(idx_hbm, table_hbm, out_hbm, idx_v, sbuf, tbuf, gsem, ssem):
        wid = lax.axis_index("s") * _NC + lax.axis_index("c")
        b0 = wid * bpw
        pltpu.sync_copy(idx_hbm.at[:, pl.ds(b0, bpw)], idx_v)

        def gather(j, buf):
            pltpu.async_copy(table_hbm.at[idx_v.at[j]], sbuf.at[buf], gsem)

        def store_desc(j, buf):
            return pltpu.make_async_copy(
                tbuf.at[buf], out_hbm.at[j, :, pl.ds(b0, bpw)], ssem
            )

        iota = lax.iota(jnp.int32, _L)
        jvecs = [jnp.int32(jj * _L) + iota for jj in range(bpw // _L)]

        def transpose(buf):
            sb = sbuf.at[buf]
            tb = tbuf.at[buf]

            def dblk(dd, carry):
                for kk in range(8):
                    d = dd * 8 + kk
                    dvec = jnp.full((_L,), d, jnp.int32)
                    for jj in range(bpw // _L):
                        vals = plsc.load_gather(sb, [jvecs[jj], dvec])
                        tb[d, pl.ds(jj * _L, _L)] = vals
                return carry

            lax.fori_loop(0, dim // 8, dblk, 0)

        for m in range(pref):
            gather(m, m)

        def outer(g, carry):
            for i in range(nbuf):
                j = nbuf * g + i

                pltpu.make_async_copy(
                    table_hbm.at[idx_v.at[j]], sbuf.at[i], gsem
                ).wait()

                @pl.when(j + pref < s)
                def _(i=i, j=j):
                    gather(j + pref, (i + pref) % nbuf)

                @pl.when(j - nbuf >= 0)
                def _(i=i, j=j):
                    store_desc(j - nbuf, i).wait()

                transpose(i)
                store_desc(j, i).start()
            return carry

        lax.fori_loop(0, s // nbuf, outer, 0)
        for j in range(s - nbuf, s):
            store_desc(j, j % nbuf).wait()

    return k(idx_t, weight)


def kernel(input_ids, weight):
    b, s = input_ids.shape
    dim = weight.shape[1]
    assert b % _NW == 0
    idx_t = input_ids.T.astype(jnp.int32)
    out_t = _sc_gather(idx_t, weight, b, s, dim)
    return jnp.transpose(out_t, (2, 0, 1))


# speedup vs baseline: 1.4452x; 1.4452x over previous
import functools

import jax
import jax.numpy as jnp
from jax import lax
from jax.experimental import pallas as pl
from jax.experimental.pallas import tpu as pltpu
from jax.experimental.pallas import tpu_sc as plsc

_INFO = plsc.get_sparse_core_info()
_NC = _INFO.num_cores
_NS = _INFO.num_subcores
_NW = _NC * _NS
_L = 16


@functools.partial(jax.jit, static_argnames=("b", "s", "dim"))
def _sc_gather(idx_t, weight, b, s, dim):
    bpw = b // _NW
    mesh = plsc.VectorSubcoreMesh(core_axis_name="c", subcore_axis_name="s")

    nbuf = 5
    pref = 3
    assert dim % _L == 0 and bpw % _L == 0

    @functools.partial(
        pl.kernel,
        out_type=jax.ShapeDtypeStruct((s, dim, b), jnp.float32),
        mesh=mesh,
        scratch_types=[
            pltpu.VMEM((s, bpw), jnp.int32),
            pltpu.VMEM((nbuf, bpw, dim), jnp.float32),
            pltpu.VMEM((nbuf, dim, bpw), jnp.float32),
            pltpu.SemaphoreType.DMA,
            pltpu.SemaphoreType.DMA,
        ],
        compiler_params=pltpu.CompilerParams(
            use_tc_tiling_on_sc=False, needs_layout_passes=False
        ),
    )
    def k(idx_hbm, table_hbm, out_hbm, idx_v, sbuf, tbuf, gsem, ssem):
        wid = lax.axis_index("s") * _NC + lax.axis_index("c")
        b0 = wid * bpw
        pltpu.sync_copy(idx_hbm.at[:, pl.ds(b0, bpw)], idx_v)

        def gather(j, buf):
            pltpu.async_copy(table_hbm.at[idx_v.at[j]], sbuf.at[buf], gsem)

        def store_desc(j, buf):
            return pltpu.make_async_copy(
                tbuf.at[buf], out_hbm.at[j, :, pl.ds(b0, bpw)], ssem
            )

        iota = lax.iota(jnp.int32, _L)
        jvecs = [jnp.int32(jj * _L) + iota for jj in range(bpw // _L)]

        def transpose(buf):
            sb = sbuf.at[buf]
            tb = tbuf.at[buf]

            @plsc.parallel_loop(0, dim, 1, unroll=8)
            def _(d):
                dvec = jnp.full((_L,), d, jnp.int32)
                for jj in range(bpw // _L):
                    vals = plsc.load_gather(sb, [jvecs[jj], dvec])
                    tb[d, pl.ds(jj * _L, _L)] = vals

        for m in range(pref):
            gather(m, m)

        def step(j, carry):
            buf = lax.rem(j, nbuf)

            pltpu.make_async_copy(
                table_hbm.at[idx_v.at[j]], sbuf.at[buf], gsem
            ).wait()

            @pl.when(j + pref < s)
            def _():
                gather(j + pref, lax.rem(j + pref, nbuf))

            @pl.when(j - nbuf >= 0)
            def _():
                store_desc(j - nbuf, buf).wait()

            transpose(buf)
            store_desc(j, buf).start()
            return carry

        lax.fori_loop(0, s, step, 0)
        for j in range(s - nbuf, s):
            store_desc(j, j % nbuf).wait()

    return k(idx_t, weight)


def kernel(input_ids, weight):
    b, s = input_ids.shape
    dim = weight.shape[1]
    assert b % _NW == 0
    idx_t = input_ids.T.astype(jnp.int32)
    out_t = _sc_gather(idx_t, weight, b, s, dim)
    return jnp.transpose(out_t, (2, 0, 1))


# speedup vs baseline: 1.4658x; 1.0143x over previous
import functools

import jax
import jax.numpy as jnp
from jax import lax
from jax.experimental import pallas as pl
from jax.experimental.pallas import tpu as pltpu
from jax.experimental.pallas import tpu_sc as plsc

_INFO = plsc.get_sparse_core_info()
_NC = _INFO.num_cores
_NS = _INFO.num_subcores
_NW = _NC * _NS
_L = 16


@functools.partial(jax.jit, static_argnames=("b", "s", "dim"))
def _sc_gather(idx_t, weight, b, s, dim):
    bpw = b // _NW
    mesh = plsc.VectorSubcoreMesh(core_axis_name="c", subcore_axis_name="s")

    nbuf = 5
    pref = 3
    assert dim % _L == 0 and bpw % _L == 0

    @functools.partial(
        pl.kernel,
        out_type=jax.ShapeDtypeStruct((s, dim, b), jnp.float32),
        mesh=mesh,
        scratch_types=[
            pltpu.VMEM((s, bpw), jnp.int32),
            pltpu.VMEM((nbuf, bpw, dim), jnp.float32),
            pltpu.VMEM((nbuf, dim, bpw), jnp.float32),
            pltpu.SemaphoreType.DMA,
            pltpu.SemaphoreType.DMA,
        ],
        compiler_params=pltpu.CompilerParams(
            use_tc_tiling_on_sc=False, needs_layout_passes=False
        ),
    )
    def k(idx_hbm, table_hbm, out_hbm, idx_v, sbuf, tbuf, gsem, ssem):
        wid = lax.axis_index("s") * _NC + lax.axis_index("c")
        b0 = wid * bpw
        pltpu.sync_copy(idx_hbm.at[:, pl.ds(b0, bpw)], idx_v)

        def gather(j, buf):
            pltpu.async_copy(table_hbm.at[idx_v.at[j]], sbuf.at[buf], gsem)

        def store_desc(j, buf):
            return pltpu.make_async_copy(
                tbuf.at[buf], out_hbm.at[j, :, pl.ds(b0, bpw)], ssem
            )

        iota = lax.iota(jnp.int32, _L)
        jvecs = [jnp.int32(jj * _L) + iota for jj in range(bpw // _L)]

        def transpose(buf):
            sb = sbuf.at[buf]
            tb = tbuf.at[buf]

            @plsc.parallel_loop(0, dim, 1, unroll=8)
            def _(d):
                dvec = jnp.full((_L,), d, jnp.int32)
                for jj in range(bpw // _L):
                    vals = plsc.load_gather(sb, [jvecs[jj], dvec])
                    tb[d, pl.ds(jj * _L, _L)] = vals

        for m in range(pref):
            gather(m, m)

        def outer(g, carry):
            for i in range(nbuf):
                j = nbuf * g + i

                pltpu.make_async_copy(
                    table_hbm.at[idx_v.at[j]], sbuf.at[i], gsem
                ).wait()

                @pl.when(j + pref < s)
                def _(i=i, j=j):
                    gather(j + pref, (i + pref) % nbuf)

                @pl.when(j - nbuf >= 0)
                def _(i=i, j=j):
                    store_desc(j - nbuf, i).wait()

                transpose(i)
                store_desc(j, i).start()
            return carry

        lax.fori_loop(0, s // nbuf, outer, 0)
        for j in range(s - nbuf, s):
            store_desc(j, j % nbuf).wait()

    return k(idx_t, weight)


def kernel(input_ids, weight):
    b, s = input_ids.shape
    dim = weight.shape[1]
    assert b % _NW == 0
    idx_t = input_ids.T.astype(jnp.int32)
    out_t = _sc_gather(idx_t, weight, b, s, dim)
    return jnp.transpose(out_t, (2, 0, 1))


# speedup vs baseline: 1.8242x; 1.2445x over previous
import functools

import jax
import jax.numpy as jnp
from jax import lax
from jax.experimental import pallas as pl
from jax.experimental.pallas import tpu as pltpu
from jax.experimental.pallas import tpu_sc as plsc

_INFO = plsc.get_sparse_core_info()
_NC = _INFO.num_cores
_NS = _INFO.num_subcores
_NW = _NC * _NS
_CHUNK = 128


@functools.partial(jax.jit, static_argnames=("n_chunks", "dim"))
def _sc_gather(idx2, weight, n_chunks, dim):
    total = _NW * n_chunks * _CHUNK
    mesh = plsc.VectorSubcoreMesh(core_axis_name="c", subcore_axis_name="s")

    nbuf = 10
    pref = 4
    assert n_chunks % nbuf == 0 and n_chunks >= nbuf

    @functools.partial(
        pl.kernel,
        out_type=jax.ShapeDtypeStruct((total, dim), jnp.float32),
        mesh=mesh,
        scratch_types=[
            pltpu.VMEM((n_chunks, _CHUNK), jnp.int32),
            pltpu.VMEM((nbuf, _CHUNK, dim), jnp.float32),
            pltpu.SemaphoreType.DMA,
            pltpu.SemaphoreType.DMA,
        ],
        compiler_params=pltpu.CompilerParams(use_tc_tiling_on_sc=False),
    )
    def k(idx_hbm, table_hbm, out_hbm, idx_v, rows_v, gsem, ssem):
        wid = lax.axis_index("s") * _NC + lax.axis_index("c")
        row0 = wid * n_chunks * _CHUNK
        pltpu.sync_copy(idx_hbm.at[pl.ds(wid * n_chunks, n_chunks)], idx_v)

        def gather(j, buf):
            pltpu.async_copy(table_hbm.at[idx_v.at[j]], rows_v.at[buf], gsem)

        def store_desc(j, buf):
            return pltpu.make_async_copy(
                rows_v.at[buf], out_hbm.at[pl.ds(row0 + j * _CHUNK, _CHUNK)], ssem
            )

        for m in range(pref):
            gather(m, m)

        def outer(g, carry):
            for i in range(nbuf):
                j = nbuf * g + i

                @pl.when(j - (nbuf - pref) >= 0)
                def _(i=i, j=j):
                    store_desc(j - (nbuf - pref), (i + pref) % nbuf).wait()

                @pl.when(j + pref < n_chunks)
                def _(i=i, j=j):
                    gather(j + pref, (i + pref) % nbuf)

                pltpu.make_async_copy(
                    table_hbm.at[idx_v.at[j]], rows_v.at[i], gsem
                ).wait()
                store_desc(j, i).start()
            return carry

        lax.fori_loop(0, n_chunks // nbuf, outer, 0)
        for j in range(n_chunks - (nbuf - pref), n_chunks):
            store_desc(j, j % nbuf).wait()

    return k(idx2, weight)


def kernel(input_ids, weight):
    b, s = input_ids.shape
    dim = weight.shape[1]
    total = b * s
    assert total % (_NW * _CHUNK) == 0
    n_chunks = total // (_NW * _CHUNK)
    idx2 = input_ids.reshape(total // _CHUNK, _CHUNK).astype(jnp.int32)
    out = _sc_gather(idx2, weight, n_chunks, dim)
    return out.reshape(b, s, dim)


# speedup vs baseline: 1.8309x; 1.0036x over previous
import functools

import jax
import jax.numpy as jnp
from jax import lax
from jax.experimental import pallas as pl
from jax.experimental.pallas import tpu as pltpu
from jax.experimental.pallas import tpu_sc as plsc

_INFO = plsc.get_sparse_core_info()
_NC = _INFO.num_cores
_NS = _INFO.num_subcores
_NW = _NC * _NS


@functools.partial(jax.jit, static_argnames=("b", "s", "dim"))
def _sc_gather(idx2, weight, b, s, dim):
    chunk = 2 * s
    n_chunks = b // (2 * _NW)
    mesh = plsc.VectorSubcoreMesh(core_axis_name="c", subcore_axis_name="s")

    nbuf = 8
    pref = 3
    assert n_chunks % nbuf == 0 and n_chunks >= nbuf

    @functools.partial(
        pl.kernel,
        out_type=jax.ShapeDtypeStruct((b, s, dim), jnp.float32),
        mesh=mesh,
        scratch_types=[
            pltpu.VMEM((n_chunks, chunk), jnp.int32),
            pltpu.VMEM((nbuf, chunk, dim), jnp.float32),
            pltpu.SemaphoreType.DMA,
            pltpu.SemaphoreType.DMA,
        ],
        compiler_params=pltpu.CompilerParams(use_tc_tiling_on_sc=False),
    )
    def k(idx_hbm, table_hbm, out_hbm, idx_v, rows_v, gsem, ssem):
        wid = lax.axis_index("s") * _NC + lax.axis_index("c")
        batch0 = wid * (2 * n_chunks)
        pltpu.sync_copy(idx_hbm.at[pl.ds(wid * n_chunks, n_chunks)], idx_v)

        def gather(j, buf):
            pltpu.async_copy(table_hbm.at[idx_v.at[j]], rows_v.at[buf], gsem)

        def store_desc(j, buf, half):
            return pltpu.make_async_copy(
                rows_v.at[buf, pl.ds(half * s, s)],
                out_hbm.at[batch0 + 2 * j + half],
                ssem,
            )

        for m in range(pref):
            gather(m, m)

        def outer(g, carry):
            for i in range(nbuf):
                j = nbuf * g + i

                @pl.when(j - (nbuf - pref) >= 0)
                def _(i=i, j=j):
                    store_desc(j - (nbuf - pref), (i + pref) % nbuf, 0).wait()
                    store_desc(j - (nbuf - pref), (i + pref) % nbuf, 1).wait()

                @pl.when(j + pref < n_chunks)
                def _(i=i, j=j):
                    gather(j + pref, (i + pref) % nbuf)

                pltpu.make_async_copy(
                    table_hbm.at[idx_v.at[j]], rows_v.at[i], gsem
                ).wait()
                store_desc(j, i, 0).start()
                store_desc(j, i, 1).start()
            return carry

        lax.fori_loop(0, n_chunks // nbuf, outer, 0)
        for j in range(n_chunks - (nbuf - pref), n_chunks):
            store_desc(j, j % nbuf, 0).wait()
            store_desc(j, j % nbuf, 1).wait()

    return k(idx2, weight)


def kernel(input_ids, weight):
    b, s = input_ids.shape
    dim = weight.shape[1]
    assert b % (2 * _NW) == 0
    idx2 = input_ids.reshape(b // 2, 2 * s).astype(jnp.int32)
    return _sc_gather(idx2, weight, b, s, dim)
